# baseline (speedup 1.0000x reference)
import jax
import jax.numpy as jnp
from jax import lax
from jax.experimental import pallas as pl
from jax.experimental.pallas import tpu as pltpu
from jax.experimental.pallas import tpu_sc as plsc

_NB = 4096
_NL = 200
_D = 64
_NW = 32
_BBLK = _NB // _NW
_BPW = _BBLK * _NL
_LCH = 2
_RNDS = _NL // _LCH
_NPAIR = _RNDS // 2


def _emb_body(x_hbm, table_hbm, out_hbm, x_v, idxp_v, rows_v, tile_v,
              s_g0, s_g1, s_o0, s_o1):
    s_gat = (s_g0, s_g1)
    s_out = (s_o0, s_o1)
    wid = lax.axis_index("s") * 2 + lax.axis_index("c")
    base = wid * _BPW

    iota = lax.iota(jnp.int32, 16)
    i200 = iota * _NL

    def build_idx(r, s):
        l0 = r * _LCH
        for j in range(_LCH):
            for c in range(8):
                v = plsc.load_gather(x_v, [i200 + (l0 + j + c * 16 * _NL)])
                idxp_v[s, pl.ds(j * 128 + c * 16, 16)] = v

    def gat_cp(s):
        return pltpu.make_async_copy(
            table_hbm.at[idxp_v.at[s]], rows_v.at[s], s_gat[s])

    def transpose(s):
        def fr_step(fr, carry):
            for fg in range(8):
                fs = jnp.full((16,), fg * 8 + fr, jnp.int32)
                for j in range(_LCH):
                    for c in range(8):
                        g = plsc.load_gather(
                            rows_v.at[s], [iota + (j * 128 + c * 16), fs])
                        tile_v[s, fg, j, 0, 0,
                               pl.ds(fr * 128 + c * 16, 16)] = g
            return carry
        lax.fori_loop(0, 8, fr_step, 0)

    def out_cp(r, s, fg):
        dst = out_hbm.at[pl.ds(r * _LCH, _LCH), pl.ds(fg, 1), pl.ds(wid, 1)]
        return pltpu.make_async_copy(tile_v.at[s, fg], dst, s_out[s])

    pltpu.sync_copy(x_hbm.at[pl.ds(base, _BPW)], x_v)
    for s in range(2):
        build_idx(s, s)
        gat_cp(s).start()

    def pair(p, carry):
        for s in range(2):
            r = p * 2 + s

            @pl.when(p >= 1)
            def _():
                for fg in range(8):
                    out_cp(r, s, fg).wait()

            gat_cp(s).wait()
            transpose(s)
            for fg in range(8):
                out_cp(r, s, fg).start()

            @pl.when(p <= _NPAIR - 2)
            def _():
                build_idx(r + 2, s)
                gat_cp(s).start()
        return carry

    lax.fori_loop(0, _NPAIR, pair, 0)

    for s in range(2):
        for fg in range(8):
            out_cp(_RNDS - 2 + s, s, fg).wait()


@jax.jit
def kernel(x, table):
    xf = x.reshape(_NB * _NL)
    mesh = plsc.VectorSubcoreMesh(core_axis_name="c", subcore_axis_name="s")
    p = pl.kernel(
        _emb_body,
        mesh=mesh,
        compiler_params=pltpu.CompilerParams(
            use_tc_tiling_on_sc=False, needs_layout_passes=False),
        out_type=jax.ShapeDtypeStruct((_NL, 8, _NW, 1024), jnp.float32),
        scratch_types=[
            pltpu.VMEM((_BPW,), jnp.int32),
            pltpu.VMEM((2, _LCH * _BBLK), jnp.int32),
            pltpu.VMEM((2, _LCH * _BBLK, _D), jnp.float32),
            pltpu.VMEM((2, 8, _LCH, 1, 1, 1024), jnp.float32),
            pltpu.SemaphoreType.DMA,
            pltpu.SemaphoreType.DMA,
            pltpu.SemaphoreType.DMA,
            pltpu.SemaphoreType.DMA,
        ],
    )(xf, table)
    out = (p.reshape(_NL, 8, _NW, 8, 128)
            .transpose(2, 4, 0, 1, 3)
            .reshape(_NB, _NL, _D))
    return out


# speedup vs baseline: 1.3331x; 1.3331x over previous
import jax
import jax.numpy as jnp
from jax import lax
from jax.experimental import pallas as pl
from jax.experimental.pallas import tpu as pltpu
from jax.experimental.pallas import tpu_sc as plsc

_NB = 4096
_NL = 200
_D = 64
_DP = 65
_NW = 32
_BBLK = _NB // _NW
_BPW = _BBLK * _NL
_LCH = 2
_CH = _LCH * _BBLK
_RNDS = _NL // _LCH
_NPAIR = _RNDS // 2


def _emb_body(x_hbm, table_hbm, out_hbm, x_v, rows_v, pad_v, tile_v,
              s_g0, s_g1, s_o0, s_o1):
    s_gat = (s_g0, s_g1)
    s_out = (s_o0, s_o1)
    wid = lax.axis_index("s") * 2 + lax.axis_index("c")

    iota = lax.iota(jnp.int32, 16)
    i65 = iota * _DP

    def gat_cp(r, s):
        return pltpu.make_async_copy(
            table_hbm.at[x_v.at[pl.ds(r * _CH, _CH)]], rows_v.at[s], s_gat[s])

    def pad_stage(s):
        def q_step(q, carry):
            fsplat = jnp.full((16,), q, jnp.int32)
            for k in range(4):
                g = plsc.load_gather(rows_v.at[s], [fsplat, iota + k * 16])
                pad_v[s, pl.ds(q * _DP + k * 16, 16)] = g
            return carry
        lax.fori_loop(0, _CH, q_step, 0)

    def transpose(s):
        def fr_step(fr, carry):
            for fg in range(8):
                f = fg * 8 + fr
                for j in range(_LCH):
                    for c in range(8):
                        g = plsc.load_gather(
                            pad_v.at[s],
                            [i65 + (f + (j * 128 + c * 16) * _DP)])
                        tile_v[s, fg, j, 0, 0,
                               pl.ds(fr * 128 + c * 16, 16)] = g
            return carry
        lax.fori_loop(0, 8, fr_step, 0)

    def out_cp(r, s, fg):
        dst = out_hbm.at[pl.ds(r * _LCH, _LCH), pl.ds(fg, 1), pl.ds(wid, 1)]
        return pltpu.make_async_copy(tile_v.at[s, fg], dst, s_out[s])

    for lg in range(25):
        pltpu.sync_copy(x_hbm.at[lg, pl.ds(wid * 1024, 1024)],
                        x_v.at[pl.ds(lg * 1024, 1024)])
    for s in range(2):
        gat_cp(s, s).start()

    def pair(p, carry):
        for s in range(2):
            r = p * 2 + s

            @pl.when(p >= 1)
            def _():
                for fg in range(8):
                    out_cp(r, s, fg).wait()

            gat_cp(r, s).wait()
            pad_stage(s)

            @pl.when(p <= _NPAIR - 2)
            def _():
                gat_cp(r + 2, s).start()

            transpose(s)
            for fg in range(8):
                out_cp(r, s, fg).start()
        return carry

    lax.fori_loop(0, _NPAIR, pair, 0)

    for s in range(2):
        for fg in range(8):
            out_cp(_RNDS - 2 + s, s, fg).wait()


@jax.jit
def kernel(x, table):
    x4 = (x.T.reshape(25, 8, _NW, 128)
           .transpose(0, 2, 1, 3)
           .reshape(25, _NW * 1024))
    mesh = plsc.VectorSubcoreMesh(core_axis_name="c", subcore_axis_name="s")
    p = pl.kernel(
        _emb_body,
        mesh=mesh,
        compiler_params=pltpu.CompilerParams(
            use_tc_tiling_on_sc=False, needs_layout_passes=False),
        out_type=jax.ShapeDtypeStruct((_NL, 8, _NW, 1024), jnp.float32),
        scratch_types=[
            pltpu.VMEM((_BPW,), jnp.int32),
            pltpu.VMEM((2, _CH, _D), jnp.float32),
            pltpu.VMEM((2, _CH * _DP), jnp.float32),
            pltpu.VMEM((2, 8, _LCH, 1, 1, 1024), jnp.float32),
            pltpu.SemaphoreType.DMA,
            pltpu.SemaphoreType.DMA,
            pltpu.SemaphoreType.DMA,
            pltpu.SemaphoreType.DMA,
        ],
    )(x4, table)
    out = (p.reshape(_NL, 8, _NW, 8, 128)
            .transpose(2, 4, 0, 1, 3)
            .reshape(_NB, _NL, _D))
    return out


# speedup vs baseline: 1.7440x; 1.3082x over previous
import jax
import jax.numpy as jnp
from jax import lax
from jax.experimental import pallas as pl
from jax.experimental.pallas import tpu as pltpu
from jax.experimental.pallas import tpu_sc as plsc

_NB = 4096
_NL = 200
_D = 64
_NW = 32
_BBLK = _NB // _NW
_BPW = _BBLK * _NL
_LCH = 4
_CH = _LCH * _BBLK
_RNDS = _NL // _LCH
_NPAIR = _RNDS // 2


def _emb_body(x_hbm, table_hbm, out_hbm, x_v, rows_v,
              s_g0, s_g1, s_o0, s_o1):
    s_gat = (s_g0, s_g1)
    s_out = (s_o0, s_o1)
    wid = lax.axis_index("s") * 2 + lax.axis_index("c")

    def gat_cp(r, s):
        return pltpu.make_async_copy(
            table_hbm.at[x_v.at[pl.ds(r * _CH, _CH)]], rows_v.at[s], s_gat[s])

    def out_cp(r, s, j):
        l = r * _LCH + j
        dst = out_hbm.at[pl.ds(wid * _BBLK, _BBLK), pl.ds(l * _D, _D)]
        return pltpu.make_async_copy(
            rows_v.at[s, pl.ds(j * _BBLK, _BBLK)], dst, s_out[s])

    for lg in range(25):
        pltpu.sync_copy(x_hbm.at[lg, pl.ds(wid * 1024, 1024)],
                        x_v.at[pl.ds(lg * 1024, 1024)])

    def pair(p, carry):
        for s in range(2):
            r = p * 2 + s

            @pl.when(p >= 1)
            def _():
                for j in range(_LCH):
                    out_cp(r, s, j).wait()

            gat_cp(r, s).start()

            if s == 0:
                @pl.when(p >= 1)
                def _():
                    gat_cp(r - 1, 1).wait()
                    for j in range(_LCH):
                        out_cp(r - 1, 1, j).start()
            else:
                gat_cp(r - 1, 0).wait()
                for j in range(_LCH):
                    out_cp(r - 1, 0, j).start()
        return carry

    lax.fori_loop(0, _NPAIR, pair, 0)

    last = _RNDS - 1
    gat_cp(last, 1).wait()
    for j in range(_LCH):
        out_cp(last, 1, j).start()
    for j in range(_LCH):
        out_cp(last - 1, 0, j).wait()
    for j in range(_LCH):
        out_cp(last, 1, j).wait()


@jax.jit
def kernel(x, table):
    x4 = (x.T.reshape(25, 8, _NW, 128)
           .transpose(0, 2, 1, 3)
           .reshape(25, _NW * 1024))
    mesh = plsc.VectorSubcoreMesh(core_axis_name="c", subcore_axis_name="s")
    p = pl.kernel(
        _emb_body,
        mesh=mesh,
        compiler_params=pltpu.CompilerParams(
            use_tc_tiling_on_sc=False, needs_layout_passes=False),
        out_type=jax.ShapeDtypeStruct((_NB, _NL * _D), jnp.float32),
        scratch_types=[
            pltpu.VMEM((_BPW,), jnp.int32),
            pltpu.VMEM((2, _CH, _D), jnp.float32),
            pltpu.SemaphoreType.DMA,
            pltpu.SemaphoreType.DMA,
            pltpu.SemaphoreType.DMA,
            pltpu.SemaphoreType.DMA,
        ],
    )(x4, table)
    return p.reshape(_NB, _NL, _D)
